# Initial kernel scaffold; baseline (speedup 1.0000x reference)
import jax
import jax.numpy as jnp
from jax.experimental import pallas as pl


def kernel(ids, token_emb, pos_emb, gamma, beta):
    raise NotImplementedError("write your pallas kernel here")


# speedup vs baseline: 1.1075x; 1.1075x over previous
import functools

import jax
import jax.numpy as jnp
from jax import lax
from jax.experimental import pallas as pl
from jax.experimental.pallas import tpu as pltpu
from jax.experimental.pallas import tpu_sc as plsc

VOCAB = 51200
D = 256
L_SEQ = 512
BATCH = 1024
N_TOK = BATCH * L_SEQ

NC = 2
NS = 16
NW = NC * NS
PER_W = N_TOK // NW
C = 64
NCH = L_SEQ // C
NSEQ = PER_W // L_SEQ
NLANE = D // 16

_EPS = 1e-5


def _lane_sum(v):
    lane = lax.iota(jnp.int32, 16)
    for m in (8, 4, 2, 1):
        perm = lax.bitwise_xor(lane, jnp.int32(m))
        v = v + lax.gather(
            v, perm[:, None],
            lax.GatherDimensionNumbers(
                offset_dims=(), collapsed_slice_dims=(0,),
                start_index_map=(0,)),
            slice_sizes=(1,),
            mode=lax.GatherScatterMode.PROMISE_IN_BOUNDS)
    return v


def _rsqrt(v16):
    i = lax.bitcast_convert_type(v16, jnp.int32)
    i = jnp.int32(0x5F3759DF) - lax.shift_right_arithmetic(i, jnp.int32(1))
    y = lax.bitcast_convert_type(i, jnp.float32)
    half = v16 * 0.5
    for _ in range(3):
        y = y * (1.5 - half * y * y)
    return y


def _sc_body(ids_hbm, tok_hbm, pos_hbm, gamma_hbm, beta_hbm, out_hbm,
             ids_v, pos_v, xbuf, obuf, gamma_v, beta_v, sem):
    wid = lax.axis_index("s") * NC + lax.axis_index("c")
    row0 = wid * (PER_W // C)
    base = wid * PER_W

    pltpu.sync_copy(ids_hbm.at[pl.ds(row0, PER_W // C)], ids_v)
    pltpu.sync_copy(gamma_hbm, gamma_v)
    pltpu.sync_copy(beta_hbm, beta_v)

    @pl.loop(0, NCH)
    def _chunk(c):
        pltpu.sync_copy(pos_hbm.at[pl.ds(c * C, C)], pos_v)

        @pl.loop(0, NSEQ)
        def _seq(s):
            j = s * NCH + c
            pltpu.async_copy(tok_hbm.at[ids_v.at[j]], xbuf, sem).wait()

            @pl.loop(0, C)
            def _row(r):
                ts = []
                acc = jnp.zeros((16,), jnp.float32)
                acc2 = jnp.zeros((16,), jnp.float32)
                for i in range(NLANE):
                    t = xbuf[r, pl.ds(i * 16, 16)] + pos_v[r, pl.ds(i * 16, 16)]
                    ts.append(t)
                    acc = acc + t
                    acc2 = acc2 + t * t
                mean_v = _lane_sum(acc) * (1.0 / D)
                ex2_v = _lane_sum(acc2) * (1.0 / D)
                var_v = ex2_v - mean_v * mean_v + _EPS
                rstd_v = _rsqrt(var_v)
                for i in range(NLANE):
                    g = gamma_v[pl.ds(i * 16, 16)]
                    b = beta_v[pl.ds(i * 16, 16)]
                    obuf[r, pl.ds(i * 16, 16)] = (ts[i] - mean_v) * (rstd_v * g) + b

            pltpu.sync_copy(obuf, out_hbm.at[pl.ds(base + j * C, C)])


@jax.jit
def _encode(ids_rows, token_emb, pos_emb, gamma, beta):
    mesh = plsc.VectorSubcoreMesh(core_axis_name="c", subcore_axis_name="s")
    f = pl.kernel(
        _sc_body,
        out_type=jax.ShapeDtypeStruct((N_TOK, D), jnp.float32),
        mesh=mesh,
        scratch_types=[
            pltpu.VMEM((PER_W // C, C), jnp.int32),
            pltpu.VMEM((C, D), jnp.float32),
            pltpu.VMEM((C, D), jnp.float32),
            pltpu.VMEM((C, D), jnp.float32),
            pltpu.VMEM((D,), jnp.float32),
            pltpu.VMEM((D,), jnp.float32),
            pltpu.SemaphoreType.DMA,
        ],
    )
    return f(ids_rows, token_emb, pos_emb, gamma, beta)


def kernel(ids, token_emb, pos_emb, gamma, beta):
    ids_rows = ids.reshape(N_TOK // C, C)
    out = _encode(ids_rows, token_emb, pos_emb, gamma, beta)
    return (out.reshape(BATCH, L_SEQ, D), ids)
